# baseline (device time: 102343 ns/iter reference)
import functools

import jax
import jax.numpy as jnp
from jax import lax
from jax.experimental import pallas as pl
from jax.experimental.pallas import tpu as pltpu

N_DEV = 8
N_EXPERTS = 16
CAPACITY = 25


def kernel(x, router_W, route_idx, expert_W):
    n_tok, d_in = x.shape
    e_per, _, d_out = expert_W.shape

    def body(x_ref, idx_ref, w_ref, out_ref, comm_ref, send_sems, recv_sems):
        my = lax.axis_index("i")
        left = lax.rem(my + N_DEV - 1, N_DEV)
        right = lax.rem(my + 1, N_DEV)

        barrier_sem = pltpu.get_barrier_semaphore()
        for nbr in (left, right):
            pl.semaphore_signal(
                barrier_sem, inc=1,
                device_id=(nbr,), device_id_type=pl.DeviceIdType.MESH,
            )
        pl.semaphore_wait(barrier_sem, 2)

        idx = idx_ref[:, :]
        eids = lax.broadcasted_iota(jnp.int32, (n_tok, N_EXPERTS), 1)
        onehot = (idx == eids).astype(jnp.float32)
        row = lax.broadcasted_iota(jnp.int32, (n_tok, n_tok), 0)
        col = lax.broadcasted_iota(jnp.int32, (n_tok, n_tok), 1)
        tril = (col <= row).astype(jnp.float32)
        cum = jnp.dot(tril, onehot, preferred_element_type=jnp.float32)
        pos = jnp.sum(cum * onehot, axis=1, keepdims=True)
        keep = pos <= CAPACITY

        acc = jnp.zeros((n_tok, d_out), jnp.float32)
        for k in range(e_per):
            ge = my * e_per + k
            mask_k = jnp.where(jnp.logical_and(idx == ge, keep), 1.0, 0.0)
            xm = x_ref[:, :] * mask_k
            acc = acc + jnp.dot(xm, w_ref[k, :, :],
                                preferred_element_type=jnp.float32)

        comm_ref[0, :, :] = acc
        out_ref[:, :] = acc

        for h in range(N_DEV - 1):
            rdma = pltpu.make_async_remote_copy(
                src_ref=comm_ref.at[h],
                dst_ref=comm_ref.at[h + 1],
                send_sem=send_sems.at[h],
                recv_sem=recv_sems.at[h + 1],
                device_id=(right,),
                device_id_type=pl.DeviceIdType.MESH,
            )
            rdma.start()
            rdma.wait()
            out_ref[:, :] = out_ref[:, :] + comm_ref[h + 1, :, :]

        @functools.partial(
            pl.run_scoped, second_barrier=pltpu.SemaphoreType.REGULAR
        )
        def _(second_barrier):
            for nbr in (left, right):
                pl.semaphore_signal(
                    second_barrier, inc=1,
                    device_id=(nbr,), device_id_type=pl.DeviceIdType.MESH,
                )
            pl.semaphore_wait(second_barrier, 2)

    return pl.pallas_call(
        body,
        out_shape=jax.ShapeDtypeStruct((n_tok, d_out), jnp.float32),
        in_specs=[
            pl.BlockSpec(memory_space=pltpu.VMEM),
            pl.BlockSpec(memory_space=pltpu.VMEM),
            pl.BlockSpec(memory_space=pltpu.VMEM),
        ],
        out_specs=pl.BlockSpec(memory_space=pltpu.VMEM),
        scratch_shapes=[
            pltpu.VMEM((N_DEV, n_tok, d_out), jnp.float32),
            pltpu.SemaphoreType.DMA((N_DEV,)),
            pltpu.SemaphoreType.DMA((N_DEV,)),
        ],
        compiler_params=pltpu.CompilerParams(collective_id=0),
    )(x, route_idx, expert_W)


# device time: 39670 ns/iter; 2.5799x vs baseline; 2.5799x over previous
import functools

import jax
import jax.numpy as jnp
from jax import lax
from jax.experimental import pallas as pl
from jax.experimental.pallas import tpu as pltpu

N_DEV = 8
N_EXPERTS = 16
CAPACITY = 25
N_CHUNKS = 8


def kernel(x, router_W, route_idx, expert_W):
    n_tok, d_in = x.shape
    e_per, _, d_out = expert_W.shape
    rows = n_tok // N_CHUNKS

    def body(x_ref, idx_ref, w_ref, out_ref,
             acc_ref, r0, r1, r2, send_sems, recv_sems):
        my = lax.axis_index("i")
        b0 = my & 1
        b1 = (my >> 1) & 1
        b2 = (my >> 2) & 1
        partners = (my ^ 4, my ^ 3, my ^ 1)

        barrier_sem = pltpu.get_barrier_semaphore()
        for p in partners:
            pl.semaphore_signal(
                barrier_sem, inc=1,
                device_id=(p,), device_id_type=pl.DeviceIdType.MESH,
            )
        pl.semaphore_wait(barrier_sem, 3)

        idx = idx_ref[:, :]
        eids = lax.broadcasted_iota(jnp.int32, (n_tok, N_EXPERTS), 1)
        onehot = (idx == eids).astype(jnp.float32)
        row = lax.broadcasted_iota(jnp.int32, (n_tok, n_tok), 0)
        col = lax.broadcasted_iota(jnp.int32, (n_tok, n_tok), 1)
        tril = (col <= row).astype(jnp.float32)
        cum = jnp.dot(tril, onehot, preferred_element_type=jnp.float32)
        pos = jnp.sum(cum * onehot, axis=1, keepdims=True)
        keep = pos <= CAPACITY

        acc = jnp.zeros((n_tok, d_out), jnp.float32)
        for k in range(e_per):
            ge = my * e_per + k
            mask_k = jnp.where(jnp.logical_and(idx == ge, keep), 1.0, 0.0)
            xm = x_ref[:, :] * mask_k
            acc = acc + jnp.dot(xm, w_ref[k, :, :],
                                preferred_element_type=jnp.float32)
        acc_ref[...] = acc.reshape(N_CHUNKS, rows, d_out)

        def exchange(step, src_slice, dst_ref_sl, partner):
            rdma = pltpu.make_async_remote_copy(
                src_ref=src_slice,
                dst_ref=dst_ref_sl,
                send_sem=send_sems.at[step],
                recv_sem=recv_sems.at[step],
                device_id=(partner,),
                device_id_type=pl.DeviceIdType.MESH,
            )
            rdma.start()
            rdma.wait()

        cur = b2 * 4
        exchange(0, acc_ref.at[pl.ds((1 - b2) * 4, 4)], r0, my ^ 4)
        acc_ref[pl.ds(cur, 4)] = acc_ref[pl.ds(cur, 4)] + r0[...]
        mine = cur + b1 * 2
        exchange(1, acc_ref.at[pl.ds(cur + (1 - b1) * 2, 2)], r1, my ^ 3)
        acc_ref[pl.ds(mine, 2)] = acc_ref[pl.ds(mine, 2)] + r1[...]
        cur = mine
        mine = cur + b0
        exchange(2, acc_ref.at[pl.ds(cur + (1 - b0), 1)], r2, my ^ 1)
        acc_ref[pl.ds(mine, 1)] = acc_ref[pl.ds(mine, 1)] + r2[...]
        cur = mine

        exchange(3, acc_ref.at[pl.ds(cur, 1)], acc_ref.at[pl.ds(cur, 1)],
                 my ^ 1)
        cur = cur - b0
        exchange(4, acc_ref.at[pl.ds(cur, 2)], acc_ref.at[pl.ds(cur, 2)],
                 my ^ 3)
        cur = cur - b1 * 2
        exchange(5, acc_ref.at[pl.ds(cur, 4)], acc_ref.at[pl.ds(cur, 4)],
                 my ^ 4)

        out_ref[:, :] = acc_ref[...].reshape(n_tok, d_out)

        @functools.partial(
            pl.run_scoped, second_barrier=pltpu.SemaphoreType.REGULAR
        )
        def _(second_barrier):
            for p in partners:
                pl.semaphore_signal(
                    second_barrier, inc=1,
                    device_id=(p,), device_id_type=pl.DeviceIdType.MESH,
                )
            pl.semaphore_wait(second_barrier, 3)

    return pl.pallas_call(
        body,
        out_shape=jax.ShapeDtypeStruct((n_tok, d_out), jnp.float32),
        in_specs=[
            pl.BlockSpec(memory_space=pltpu.VMEM),
            pl.BlockSpec(memory_space=pltpu.VMEM),
            pl.BlockSpec(memory_space=pltpu.VMEM),
        ],
        out_specs=pl.BlockSpec(memory_space=pltpu.VMEM),
        scratch_shapes=[
            pltpu.VMEM((N_CHUNKS, rows, d_out), jnp.float32),
            pltpu.VMEM((4, rows, d_out), jnp.float32),
            pltpu.VMEM((2, rows, d_out), jnp.float32),
            pltpu.VMEM((1, rows, d_out), jnp.float32),
            pltpu.SemaphoreType.DMA((6,)),
            pltpu.SemaphoreType.DMA((6,)),
        ],
        compiler_params=pltpu.CompilerParams(collective_id=0),
    )(x, route_idx, expert_W)


# device time: 18961 ns/iter; 5.3976x vs baseline; 2.0922x over previous
import functools

import jax
import jax.numpy as jnp
from jax import lax
from jax.experimental import pallas as pl
from jax.experimental.pallas import tpu as pltpu

N_DEV = 8
N_EXPERTS = 16
CAPACITY = 25
CAP_PAD = 32


def kernel(x, router_W, route_idx, expert_W):
    n_tok, d_in = x.shape
    e_per, _, d_out = expert_W.shape
    blk = e_per * CAP_PAD
    n_slots = N_EXPERTS * CAP_PAD

    def body(x_ref, idx_ref, w_ref, out_ref, gbuf, send_sems, recv_sems):
        my = lax.axis_index("i")

        barrier_sem = pltpu.get_barrier_semaphore()
        for m in range(1, N_DEV):
            pl.semaphore_signal(
                barrier_sem, inc=1,
                device_id=(my ^ m,), device_id_type=pl.DeviceIdType.MESH,
            )
        pl.semaphore_wait(barrier_sem, N_DEV - 1)

        idx = idx_ref[:, :]
        eids = lax.broadcasted_iota(jnp.int32, (n_tok, N_EXPERTS), 1)
        onehot = (idx == eids).astype(jnp.float32)
        row = lax.broadcasted_iota(jnp.int32, (n_tok, n_tok), 0)
        col = lax.broadcasted_iota(jnp.int32, (n_tok, n_tok), 1)
        tril = (col <= row).astype(jnp.float32)
        cum = jnp.dot(tril, onehot, preferred_element_type=jnp.float32)
        pos = jnp.sum(cum * onehot, axis=1, keepdims=True)

        s_id = lax.broadcasted_iota(jnp.int32, (n_tok, n_slots), 1)
        e_of_s = s_id // CAP_PAD
        c_of_s = s_id % CAP_PAD
        Pt = jnp.where(
            (idx == e_of_s)
            & (pos == (c_of_s + 1).astype(jnp.float32))
            & (pos <= CAPACITY),
            1.0,
            0.0,
        )

        sl_id = lax.broadcasted_iota(jnp.int32, (n_tok, blk), 1)
        e_loc = my * e_per + sl_id // CAP_PAD
        c_loc = sl_id % CAP_PAD
        P_local_t = jnp.where(
            (idx == e_loc)
            & (pos == (c_loc + 1).astype(jnp.float32))
            & (pos <= CAPACITY),
            1.0,
            0.0,
        )
        cx = lax.dot_general(
            P_local_t, x_ref[:, :], (((0,), (0,)), ((), ())),
            preferred_element_type=jnp.float32,
        )
        y0 = jnp.dot(cx[:CAP_PAD], w_ref[0, :, :],
                     preferred_element_type=jnp.float32)
        y1 = jnp.dot(cx[CAP_PAD:], w_ref[1, :, :],
                     preferred_element_type=jnp.float32)
        gbuf[pl.ds(my, 1)] = jnp.concatenate([y0, y1], axis=0).reshape(
            1, blk, d_out
        )

        rdmas = []
        for m in range(1, N_DEV):
            rdma = pltpu.make_async_remote_copy(
                src_ref=gbuf.at[pl.ds(my, 1)],
                dst_ref=gbuf.at[pl.ds(my, 1)],
                send_sem=send_sems.at[m - 1],
                recv_sem=recv_sems.at[m - 1],
                device_id=(my ^ m,),
                device_id_type=pl.DeviceIdType.MESH,
            )
            rdma.start()
            rdmas.append(rdma)
        for rdma in rdmas:
            rdma.wait()

        y_all = gbuf[...].reshape(n_slots, d_out)
        out_ref[:, :] = jnp.dot(
            Pt, y_all, preferred_element_type=jnp.float32
        )

        @functools.partial(
            pl.run_scoped, second_barrier=pltpu.SemaphoreType.REGULAR
        )
        def _(second_barrier):
            for m in range(1, N_DEV):
                pl.semaphore_signal(
                    second_barrier, inc=1,
                    device_id=(my ^ m,), device_id_type=pl.DeviceIdType.MESH,
                )
            pl.semaphore_wait(second_barrier, N_DEV - 1)

    return pl.pallas_call(
        body,
        out_shape=jax.ShapeDtypeStruct((n_tok, d_out), jnp.float32),
        in_specs=[
            pl.BlockSpec(memory_space=pltpu.VMEM),
            pl.BlockSpec(memory_space=pltpu.VMEM),
            pl.BlockSpec(memory_space=pltpu.VMEM),
        ],
        out_specs=pl.BlockSpec(memory_space=pltpu.VMEM),
        scratch_shapes=[
            pltpu.VMEM((N_DEV, blk, d_out), jnp.float32),
            pltpu.SemaphoreType.DMA((N_DEV - 1,)),
            pltpu.SemaphoreType.DMA((N_DEV - 1,)),
        ],
        compiler_params=pltpu.CompilerParams(collective_id=0),
    )(x, route_idx, expert_W)


# device time: 17831 ns/iter; 5.7396x vs baseline; 1.0634x over previous
import functools

import jax
import jax.numpy as jnp
from jax import lax
from jax.experimental import pallas as pl
from jax.experimental.pallas import tpu as pltpu

N_DEV = 8
N_EXPERTS = 16
CAPACITY = 25
CAP_PAD = 32


def kernel(x, router_W, route_idx, expert_W):
    n_tok, d_in = x.shape
    e_per, _, d_out = expert_W.shape
    blk = e_per * CAP_PAD
    n_slots = N_EXPERTS * CAP_PAD

    def body(x_ref, idx_ref, w_ref, out_ref, gbuf, send_sems, recv_sems):
        my = lax.axis_index("i")

        barrier_sem = pltpu.get_barrier_semaphore()
        for m in range(1, N_DEV):
            pl.semaphore_signal(
                barrier_sem, inc=1,
                device_id=(my ^ m,), device_id_type=pl.DeviceIdType.MESH,
            )
        pl.semaphore_wait(barrier_sem, N_DEV - 1)

        idx = idx_ref[:, :]
        eids = lax.broadcasted_iota(jnp.int32, (n_tok, N_EXPERTS), 1)
        onehot = (idx == eids).astype(jnp.float32)
        row = lax.broadcasted_iota(jnp.int32, (n_tok, n_tok), 0)
        col = lax.broadcasted_iota(jnp.int32, (n_tok, n_tok), 1)
        tril = (col <= row).astype(jnp.float32)
        cum = jnp.dot(tril, onehot, preferred_element_type=jnp.float32)
        pos = jnp.sum(cum * onehot, axis=1, keepdims=True)

        sl_id = lax.broadcasted_iota(jnp.int32, (n_tok, blk), 1)
        e_loc = my * e_per + sl_id // CAP_PAD
        c_loc = sl_id % CAP_PAD
        P_local_t = jnp.where(
            (idx == e_loc)
            & (pos == (c_loc + 1).astype(jnp.float32))
            & (pos <= CAPACITY),
            1.0,
            0.0,
        )
        cx = lax.dot_general(
            P_local_t, x_ref[:, :], (((0,), (0,)), ((), ())),
            preferred_element_type=jnp.float32,
        )
        y0 = jnp.dot(cx[:CAP_PAD], w_ref[0, :, :],
                     preferred_element_type=jnp.float32)
        y1 = jnp.dot(cx[CAP_PAD:], w_ref[1, :, :],
                     preferred_element_type=jnp.float32)
        gbuf[pl.ds(my, 1)] = (
            jnp.concatenate([y0, y1], axis=0)
            .astype(jnp.bfloat16)
            .reshape(1, blk, d_out)
        )

        rdmas = []
        for m in range(1, N_DEV):
            rdma = pltpu.make_async_remote_copy(
                src_ref=gbuf.at[pl.ds(my, 1)],
                dst_ref=gbuf.at[pl.ds(my, 1)],
                send_sem=send_sems.at[m - 1],
                recv_sem=recv_sems.at[m - 1],
                device_id=(my ^ m,),
                device_id_type=pl.DeviceIdType.MESH,
            )
            rdma.start()
            rdmas.append(rdma)

        s_id = lax.broadcasted_iota(jnp.int32, (n_tok, n_slots), 1)
        e_of_s = s_id // CAP_PAD
        c_of_s = s_id % CAP_PAD
        Pt = jnp.where(
            (idx == e_of_s)
            & (pos == (c_of_s + 1).astype(jnp.float32))
            & (pos <= CAPACITY),
            1.0,
            0.0,
        ).astype(jnp.bfloat16)

        for rdma in rdmas:
            rdma.wait()

        y_all = gbuf[...].reshape(n_slots, d_out)
        out_ref[:, :] = jnp.dot(
            Pt, y_all, preferred_element_type=jnp.float32
        )

        @functools.partial(
            pl.run_scoped, second_barrier=pltpu.SemaphoreType.REGULAR
        )
        def _(second_barrier):
            for m in range(1, N_DEV):
                pl.semaphore_signal(
                    second_barrier, inc=1,
                    device_id=(my ^ m,), device_id_type=pl.DeviceIdType.MESH,
                )
            pl.semaphore_wait(second_barrier, N_DEV - 1)

    return pl.pallas_call(
        body,
        out_shape=jax.ShapeDtypeStruct((n_tok, d_out), jnp.float32),
        in_specs=[
            pl.BlockSpec(memory_space=pltpu.VMEM),
            pl.BlockSpec(memory_space=pltpu.VMEM),
            pl.BlockSpec(memory_space=pltpu.VMEM),
        ],
        out_specs=pl.BlockSpec(memory_space=pltpu.VMEM),
        scratch_shapes=[
            pltpu.VMEM((N_DEV, blk, d_out), jnp.bfloat16),
            pltpu.SemaphoreType.DMA((N_DEV - 1,)),
            pltpu.SemaphoreType.DMA((N_DEV - 1,)),
        ],
        compiler_params=pltpu.CompilerParams(collective_id=0),
    )(x, route_idx, expert_W)


# device time: 14694 ns/iter; 6.9650x vs baseline; 1.2135x over previous
import jax
import jax.numpy as jnp
from jax import lax
from jax.experimental import pallas as pl
from jax.experimental.pallas import tpu as pltpu

N_DEV = 8
N_EXPERTS = 16
CAPACITY = 25
CAP_PAD = 32


def kernel(x, router_W, route_idx, expert_W):
    n_tok, d_in = x.shape
    e_per, _, d_out = expert_W.shape
    blk = e_per * CAP_PAD
    n_slots = N_EXPERTS * CAP_PAD

    def body(x_ref, idx_ref, w_ref, out_ref, gbuf, send_sems, recv_sems):
        my = lax.axis_index("i")

        barrier_sem = pltpu.get_barrier_semaphore()
        for m in range(1, N_DEV):
            pl.semaphore_signal(
                barrier_sem, inc=1,
                device_id=(my ^ m,), device_id_type=pl.DeviceIdType.MESH,
            )

        idx = idx_ref[:, :]
        eids = lax.broadcasted_iota(jnp.int32, (n_tok, N_EXPERTS), 1)
        onehot = (idx == eids).astype(jnp.float32)
        row = lax.broadcasted_iota(jnp.int32, (n_tok, n_tok), 0)
        col = lax.broadcasted_iota(jnp.int32, (n_tok, n_tok), 1)
        tril = (col <= row).astype(jnp.float32)
        cum = jnp.dot(tril, onehot, preferred_element_type=jnp.float32)
        pos = jnp.sum(cum * onehot, axis=1, keepdims=True)
        pos_i = pos.astype(jnp.int32)

        slot_t = jnp.where(
            pos_i <= CAPACITY, idx * CAP_PAD + pos_i - 1, -1
        )

        lsl = slot_t - my * blk
        P_local_t = (
            lax.broadcasted_iota(jnp.int32, (n_tok, blk), 1) == lsl
        ).astype(jnp.float32)
        cx = lax.dot_general(
            P_local_t, x_ref[:, :], (((0,), (0,)), ((), ())),
            preferred_element_type=jnp.float32,
        )
        y0 = jnp.dot(cx[:CAP_PAD], w_ref[0, :, :],
                     preferred_element_type=jnp.float32)
        y1 = jnp.dot(cx[CAP_PAD:], w_ref[1, :, :],
                     preferred_element_type=jnp.float32)
        gbuf[pl.ds(my, 1)] = (
            jnp.concatenate([y0, y1], axis=0)
            .astype(jnp.bfloat16)
            .reshape(1, blk, d_out)
        )

        pl.semaphore_wait(barrier_sem, N_DEV - 1)

        rdmas = []
        for m in range(1, N_DEV):
            rdma = pltpu.make_async_remote_copy(
                src_ref=gbuf.at[pl.ds(my, 1)],
                dst_ref=gbuf.at[pl.ds(my, 1)],
                send_sem=send_sems.at[m - 1],
                recv_sem=recv_sems.at[m - 1],
                device_id=(my ^ m,),
                device_id_type=pl.DeviceIdType.MESH,
            )
            rdma.start()
            rdmas.append(rdma)

        s_id = lax.broadcasted_iota(jnp.int32, (n_tok, n_slots), 1)
        Pt = (s_id == slot_t).astype(jnp.float32).astype(jnp.bfloat16)

        for rdma in rdmas:
            rdma.wait()

        y_all = gbuf[...].reshape(n_slots, d_out)
        out_ref[:, :] = jnp.dot(
            Pt, y_all, preferred_element_type=jnp.float32
        )

    return pl.pallas_call(
        body,
        out_shape=jax.ShapeDtypeStruct((n_tok, d_out), jnp.float32),
        in_specs=[
            pl.BlockSpec(memory_space=pltpu.VMEM),
            pl.BlockSpec(memory_space=pltpu.VMEM),
            pl.BlockSpec(memory_space=pltpu.VMEM),
        ],
        out_specs=pl.BlockSpec(memory_space=pltpu.VMEM),
        scratch_shapes=[
            pltpu.VMEM((N_DEV, blk, d_out), jnp.bfloat16),
            pltpu.SemaphoreType.DMA((N_DEV - 1,)),
            pltpu.SemaphoreType.DMA((N_DEV - 1,)),
        ],
        compiler_params=pltpu.CompilerParams(collective_id=0),
    )(x, route_idx, expert_W)
